# baseline (speedup 1.0000x reference)
import functools

import jax
import jax.numpy as jnp
from jax import lax
from jax.experimental import pallas as pl
from jax.experimental.pallas import tpu as pltpu
from jax.experimental.pallas import tpu_sc as plsc

T = 2048
H = 16
CH = 4
L = 16
PAD = 16
NCHUNK = (2 * T) // L
WORKERS_PER_HEAD = 32 // CH
GROUPS_PER_WORKER = (T // 8) // WORKERS_PER_HEAD
ROWSTRIDE = 2 * T + PAD


def _sc_body(tr_hbm, off_hbm, out_hbm, tr_v, s8_v, off_v, sem):
    c = lax.axis_index("c")
    s = lax.axis_index("s")
    wid = s * 2 + c
    head = wid // WORKERS_PER_HEAD
    part = wid % WORKERS_PER_HEAD

    pltpu.sync_copy(tr_hbm.at[pl.ds(head * (2 * T), 2 * T)],
                    tr_v.at[pl.ds(PAD, 2 * T)])
    pltpu.sync_copy(off_hbm, off_v)

    x = off_v[...]
    e = jnp.exp(x + x)
    bo = (1.0 - 2.0 / (e + 1.0)) * 0.5
    pos = bo >= 0.0

    def chunk(i, carry):
        k0 = i * L
        kk = lax.iota(jnp.int32, L) + k0
        d = 4094.0 - kk.astype(jnp.float32)
        adj = jnp.clip(d + bo, 0.0, 4094.0)
        fl = adj.astype(jnp.int32).astype(jnp.float32)
        w = adj - fl
        t0 = tr_v[pl.ds(PAD + k0, L)]
        tl = tr_v[pl.ds(PAD + k0 + 1, L)]
        tm = tr_v[pl.ds(PAD + k0 - 1, L)]
        a = jnp.where(pos, t0, tl)
        b = jnp.where(pos, tm, t0)
        v = a * (1.0 - w) + b * w
        for r in range(8):
            s8_v[pl.ds(r * ROWSTRIDE + PAD + k0 + r - 7, L)] = v
        return carry

    lax.fori_loop(0, NCHUNK, chunk, 0)

    g0 = part * GROUPS_PER_WORKER
    row_base = head * (T * T)

    def fire(j, carry):
        g = g0 + j
        start = PAD + 2040 - 8 * g
        for r in range(8):
            pltpu.async_copy(
                s8_v.at[pl.ds(r * ROWSTRIDE + start, T)],
                out_hbm.at[pl.ds(row_base + (8 * g + r) * T, T)],
                sem,
            )
        return carry

    lax.fori_loop(0, GROUPS_PER_WORKER, fire, 0)

    def drain(j, carry):
        g = g0 + j
        start = PAD + 2040 - 8 * g
        for r in range(8):
            pltpu.make_async_copy(
                s8_v.at[pl.ds(r * ROWSTRIDE + start, T)],
                out_hbm.at[pl.ds(row_base + (8 * g + r) * T, T)],
                sem,
            ).wait()
        return carry

    lax.fori_loop(0, GROUPS_PER_WORKER, drain, 0)


_sc_call = functools.partial(
    pl.kernel,
    out_type=jax.ShapeDtypeStruct((CH * T * T,), jnp.float32),
    mesh=plsc.VectorSubcoreMesh(core_axis_name="c", subcore_axis_name="s"),
    scratch_types=[
        pltpu.VMEM((2 * T + 2 * PAD, ), jnp.float32),
        pltpu.VMEM((8 * ROWSTRIDE,), jnp.float32),
        pltpu.VMEM((L,), jnp.float32),
        pltpu.SemaphoreType.DMA,
    ],
)(_sc_body)


def kernel(relative_position_bias_table, learnable_offset):
    tbl = relative_position_bias_table
    trp = jnp.asarray(
        jnp.concatenate([tbl[::-1], tbl[:1]], axis=0).T, jnp.float32)
    off16 = jnp.broadcast_to(learnable_offset.astype(jnp.float32), (L,))
    parts = []
    for h0 in range(0, H, CH):
        flat = _sc_call(trp[h0:h0 + CH].reshape(-1), off16)
        parts.append(flat.reshape(1, CH, T, T))
    return jnp.concatenate(parts, axis=1)


# speedup vs baseline: 1.0029x; 1.0029x over previous
import functools

import jax
import jax.numpy as jnp
from jax import lax
from jax.experimental import pallas as pl
from jax.experimental.pallas import tpu as pltpu
from jax.experimental.pallas import tpu_sc as plsc

T = 2048
H = 16
CH = 8
L = 16
PAD = 16
NCHUNK = (2 * T) // L
WORKERS_PER_HEAD = 32 // CH
GROUPS_PER_WORKER = (T // 8) // WORKERS_PER_HEAD
ROWSTRIDE = 2 * T + PAD


def _sc_body(tr_hbm, off_hbm, out_hbm, tr_v, s8_v, off_v, sem):
    c = lax.axis_index("c")
    s = lax.axis_index("s")
    wid = s * 2 + c
    head = wid // WORKERS_PER_HEAD
    part = wid % WORKERS_PER_HEAD

    pltpu.sync_copy(tr_hbm.at[pl.ds(head * (2 * T), 2 * T)],
                    tr_v.at[pl.ds(PAD, 2 * T)])
    pltpu.sync_copy(off_hbm, off_v)

    x = off_v[...]
    e = jnp.exp(x + x)
    bo = (1.0 - 2.0 / (e + 1.0)) * 0.5
    pos = bo >= 0.0

    def chunk(i, carry):
        k0 = i * L
        kk = lax.iota(jnp.int32, L) + k0
        d = 4094.0 - kk.astype(jnp.float32)
        adj = jnp.clip(d + bo, 0.0, 4094.0)
        fl = adj.astype(jnp.int32).astype(jnp.float32)
        w = adj - fl
        t0 = tr_v[pl.ds(PAD + k0, L)]
        tl = tr_v[pl.ds(PAD + k0 + 1, L)]
        tm = tr_v[pl.ds(PAD + k0 - 1, L)]
        a = jnp.where(pos, t0, tl)
        b = jnp.where(pos, tm, t0)
        v = a * (1.0 - w) + b * w
        for r in range(8):
            s8_v[pl.ds(r * ROWSTRIDE + PAD + k0 + r - 7, L)] = v
        return carry

    lax.fori_loop(0, NCHUNK, chunk, 0)

    g0 = part * GROUPS_PER_WORKER
    row_base = head * (T * T)

    def fire(j, carry):
        g = g0 + j
        start = PAD + 2040 - 8 * g
        for r in range(8):
            pltpu.async_copy(
                s8_v.at[pl.ds(r * ROWSTRIDE + start, T)],
                out_hbm.at[pl.ds(row_base + (8 * g + r) * T, T)],
                sem,
            )
        return carry

    lax.fori_loop(0, GROUPS_PER_WORKER, fire, 0)

    def drain(j, carry):
        g = g0 + j
        start = PAD + 2040 - 8 * g
        for r in range(8):
            pltpu.make_async_copy(
                s8_v.at[pl.ds(r * ROWSTRIDE + start, T)],
                out_hbm.at[pl.ds(row_base + (8 * g + r) * T, T)],
                sem,
            ).wait()
        return carry

    lax.fori_loop(0, GROUPS_PER_WORKER, drain, 0)


_sc_call = functools.partial(
    pl.kernel,
    out_type=jax.ShapeDtypeStruct((CH * T * T,), jnp.float32),
    mesh=plsc.VectorSubcoreMesh(core_axis_name="c", subcore_axis_name="s"),
    scratch_types=[
        pltpu.VMEM((2 * T + 2 * PAD, ), jnp.float32),
        pltpu.VMEM((8 * ROWSTRIDE,), jnp.float32),
        pltpu.VMEM((L,), jnp.float32),
        pltpu.SemaphoreType.DMA,
    ],
)(_sc_body)


def kernel(relative_position_bias_table, learnable_offset):
    tbl = relative_position_bias_table
    trp = jnp.asarray(
        jnp.concatenate([tbl[::-1], tbl[:1]], axis=0).T, jnp.float32)
    off16 = jnp.broadcast_to(learnable_offset.astype(jnp.float32), (L,))
    parts = []
    for h0 in range(0, H, CH):
        flat = _sc_call(trp[h0:h0 + CH].reshape(-1), off16)
        parts.append(flat.reshape(1, CH, T, T))
    return jnp.concatenate(parts, axis=1)


# speedup vs baseline: 2.6685x; 2.6609x over previous
import functools

import jax
import jax.numpy as jnp
from jax import lax
from jax.experimental import pallas as pl
from jax.experimental.pallas import tpu as pltpu
from jax.experimental.pallas import tpu_sc as plsc

T = 2048
H = 16
L = 16
PAD = 16
KTOT = 2 * T
NCHUNK = KTOT // L
ROWSTRIDE = KTOT + PAD
W = 3968
NR = 128
BR = 256
NB = T // BR


def _sc_body(tr_hbm, off_hbm, s128_hbm, tr_v, s8_v, off_v, sem):
    c = lax.axis_index("c")
    s = lax.axis_index("s")
    head = s

    pltpu.sync_copy(tr_hbm.at[pl.ds(head * KTOT, KTOT)],
                    tr_v.at[pl.ds(PAD, KTOT)])
    pltpu.sync_copy(off_hbm, off_v)
    zero = jnp.zeros((L,), jnp.float32)
    tr_v[pl.ds(0, L)] = zero
    tr_v[pl.ds(PAD + KTOT, L)] = zero

    x = off_v[...]
    e = jnp.exp(x + x)
    bo = (1.0 - 2.0 / (e + 1.0)) * 0.5
    pos = bo >= 0.0

    def chunk(i, carry):
        k0 = i * L
        kk = lax.iota(jnp.int32, L) + k0
        d = 4094.0 - kk.astype(jnp.float32)
        adj = jnp.clip(d + bo, 0.0, 4094.0)
        fl = adj.astype(jnp.int32).astype(jnp.float32)
        w = adj - fl
        t0 = tr_v[pl.ds(PAD + k0, L)]
        tl = tr_v[pl.ds(PAD + k0 + 1, L)]
        tm = tr_v[pl.ds(PAD + k0 - 1, L)]
        a = jnp.where(pos, t0, tl)
        b = jnp.where(pos, tm, t0)
        v = a * (1.0 - w) + b * w
        for q in range(8):
            s8_v[pl.ds(q * ROWSTRIDE + PAD + k0 + q - 7, L)] = v
        return carry

    lax.fori_loop(0, NCHUNK, chunk, 0)

    r0 = c * (NR // 2)

    def fire(j, carry):
        r = r0 + j
        a = r // 8
        q = r % 8
        pltpu.async_copy(
            s8_v.at[pl.ds(q * ROWSTRIDE + PAD + 120 - 8 * a, W)],
            s128_hbm.at[pl.ds((head * NR + r) * W, W)],
            sem,
        )
        return carry

    lax.fori_loop(0, NR // 2, fire, 0)

    def drain(j, carry):
        r = r0 + j
        a = r // 8
        q = r % 8
        pltpu.make_async_copy(
            s8_v.at[pl.ds(q * ROWSTRIDE + PAD + 120 - 8 * a, W)],
            s128_hbm.at[pl.ds((head * NR + r) * W, W)],
            sem,
        ).wait()
        return carry

    lax.fori_loop(0, NR // 2, drain, 0)


_sc_interp = functools.partial(
    pl.kernel,
    out_type=jax.ShapeDtypeStruct((H * NR * W,), jnp.float32),
    mesh=plsc.VectorSubcoreMesh(core_axis_name="c", subcore_axis_name="s"),
    scratch_types=[
        pltpu.VMEM((KTOT + 2 * PAD,), jnp.float32),
        pltpu.VMEM((8 * ROWSTRIDE,), jnp.float32),
        pltpu.VMEM((L,), jnp.float32),
        pltpu.SemaphoreType.DMA,
    ],
)(_sc_body)


def _tc_body(s_ref, out_ref):
    b = pl.program_id(1)
    for u in range(BR // NR):
        off = pl.multiple_of(1920 - 256 * b - NR * u, NR)
        out_ref[0, 0, pl.ds(NR * u, NR), :] = s_ref[0, :, pl.ds(off, T)]


def _tc_expand(s128):
    return pl.pallas_call(
        _tc_body,
        grid=(H, NB),
        in_specs=[pl.BlockSpec((1, NR, W), lambda h, b: (h, 0, 0))],
        out_specs=pl.BlockSpec((1, 1, BR, T), lambda h, b: (0, h, b, 0)),
        out_shape=jax.ShapeDtypeStruct((1, H, T, T), jnp.float32),
    )(s128)


def kernel(relative_position_bias_table, learnable_offset):
    tbl = relative_position_bias_table
    trp = jnp.asarray(
        jnp.concatenate([tbl[::-1], tbl[:1]], axis=0).T, jnp.float32)
    off16 = jnp.broadcast_to(learnable_offset.astype(jnp.float32), (L,))
    s128 = _sc_interp(trp.reshape(-1), off16)
    return _tc_expand(s128.reshape(H, NR, W))


# speedup vs baseline: 2.9607x; 1.1095x over previous
import functools

import jax
import jax.numpy as jnp
from jax import lax
from jax.experimental import pallas as pl
from jax.experimental.pallas import tpu as pltpu
from jax.experimental.pallas import tpu_sc as plsc

T = 2048
H = 16
L = 16
PAD = 16
KTOT = 2 * T
NCHUNK = KTOT // L
ROWSTRIDE = KTOT + PAD
W = 3968
NR = 128
BR = 256
NB = T // BR


def _sc_body(tr_hbm, off_hbm, s128_hbm, tr_v, s8_v, off_v, sem):
    c = lax.axis_index("c")
    s = lax.axis_index("s")
    head = s

    pltpu.sync_copy(tr_hbm.at[pl.ds(head * KTOT, KTOT)],
                    tr_v.at[pl.ds(PAD, KTOT)])
    pltpu.sync_copy(off_hbm, off_v)
    zero = jnp.zeros((L,), jnp.float32)
    tr_v[pl.ds(0, L)] = zero
    tr_v[pl.ds(PAD + KTOT, L)] = zero

    x = off_v[...]
    e = jnp.exp(x + x)
    bo = (1.0 - 2.0 / (e + 1.0)) * 0.5
    pos = bo >= 0.0

    def chunk(i, carry):
        k0 = i * L
        kk = lax.iota(jnp.int32, L) + k0
        d = 4094.0 - kk.astype(jnp.float32)
        adj = jnp.clip(d + bo, 0.0, 4094.0)
        fl = adj.astype(jnp.int32).astype(jnp.float32)
        w = adj - fl
        t0 = tr_v[pl.ds(PAD + k0, L)]
        tl = tr_v[pl.ds(PAD + k0 + 1, L)]
        tm = tr_v[pl.ds(PAD + k0 - 1, L)]
        a = jnp.where(pos, t0, tl)
        b = jnp.where(pos, tm, t0)
        v = a * (1.0 - w) + b * w
        for q in range(8):
            s8_v[pl.ds(q * ROWSTRIDE + PAD + k0 + q - 7, L)] = v
        return carry

    lax.fori_loop(0, NCHUNK, chunk, 0)

    r0 = c * (NR // 2)

    def fire(j, carry):
        r = r0 + j
        a = r // 8
        q = r % 8
        pltpu.async_copy(
            s8_v.at[pl.ds(q * ROWSTRIDE + PAD + 120 - 8 * a, W)],
            s128_hbm.at[pl.ds((head * NR + r) * W, W)],
            sem,
        )
        return carry

    lax.fori_loop(0, NR // 2, fire, 0)

    def drain(j, carry):
        r = r0 + j
        a = r // 8
        q = r % 8
        pltpu.make_async_copy(
            s8_v.at[pl.ds(q * ROWSTRIDE + PAD + 120 - 8 * a, W)],
            s128_hbm.at[pl.ds((head * NR + r) * W, W)],
            sem,
        ).wait()
        return carry

    lax.fori_loop(0, NR // 2, drain, 0)


_sc_interp = functools.partial(
    pl.kernel,
    out_type=jax.ShapeDtypeStruct((H * NR * W,), jnp.float32),
    mesh=plsc.VectorSubcoreMesh(core_axis_name="c", subcore_axis_name="s"),
    scratch_types=[
        pltpu.VMEM((KTOT + 2 * PAD,), jnp.float32),
        pltpu.VMEM((8 * ROWSTRIDE,), jnp.float32),
        pltpu.VMEM((L,), jnp.float32),
        pltpu.SemaphoreType.DMA,
    ],
)(_sc_body)


def _tc_body(s_ref, out_ref, sem):
    h = pl.program_id(0)
    copies = []
    for g in range(T // NR):
        cp = pltpu.make_async_copy(
            s_ref.at[0, :, pl.ds(1920 - NR * g, T)],
            out_ref.at[0, h, pl.ds(NR * g, NR), :],
            sem,
        )
        cp.start()
        copies.append(cp)
    for cp in copies:
        cp.wait()


def _tc_expand(s128):
    return pl.pallas_call(
        _tc_body,
        grid=(H,),
        in_specs=[pl.BlockSpec((1, NR, W), lambda h: (h, 0, 0))],
        out_specs=pl.BlockSpec(memory_space=pltpu.MemorySpace.HBM),
        out_shape=jax.ShapeDtypeStruct((1, H, T, T), jnp.float32),
        scratch_shapes=[pltpu.SemaphoreType.DMA],
    )(s128)


def kernel(relative_position_bias_table, learnable_offset):
    tbl = relative_position_bias_table
    trp = jnp.asarray(
        jnp.concatenate([tbl[::-1], tbl[:1]], axis=0).T, jnp.float32)
    off16 = jnp.broadcast_to(learnable_offset.astype(jnp.float32), (L,))
    s128 = _sc_interp(trp.reshape(-1), off16)
    return _tc_expand(s128.reshape(H, NR, W))


# speedup vs baseline: 3.1640x; 1.0687x over previous
import functools

import jax
import jax.numpy as jnp
from jax import lax
from jax.experimental import pallas as pl
from jax.experimental.pallas import tpu as pltpu
from jax.experimental.pallas import tpu_sc as plsc

T = 2048
H = 16
L = 16
PAD = 16
KTOT = 2 * T
NCHUNK = KTOT // L
ROWSTRIDE = KTOT + PAD
W = 3968
NR = 128
BR = 256
NB = T // BR


def _sc_body(tr_hbm, off_hbm, s128_hbm, tr_v, s8_v, off_v, sem):
    c = lax.axis_index("c")
    s = lax.axis_index("s")
    head = s

    pltpu.sync_copy(tr_hbm.at[pl.ds(head * KTOT, KTOT)],
                    tr_v.at[pl.ds(PAD, KTOT)])
    pltpu.sync_copy(off_hbm, off_v)
    tr_v[pl.ds(0, L)] = lax.rev(tr_v[pl.ds(PAD, L)], (0,))

    x = off_v[...]
    e = jnp.exp(x + x)
    bo = (1.0 - 2.0 / (e + 1.0)) * 0.5
    pos = bo >= 0.0

    def chunk(i, carry):
        k0 = i * L
        kk = lax.iota(jnp.int32, L) + k0
        d = 4094.0 - kk.astype(jnp.float32)
        adj = jnp.clip(d + bo, 0.0, 4094.0)
        fl = adj.astype(jnp.int32).astype(jnp.float32)
        w = adj - fl
        t0 = lax.rev(tr_v[pl.ds(PAD + 4079 - k0, L)], (0,))
        tl = lax.rev(tr_v[pl.ds(PAD + 4078 - k0, L)], (0,))
        tm = lax.rev(tr_v[pl.ds(PAD + 4080 - k0, L)], (0,))
        a = jnp.where(pos, t0, tl)
        b = jnp.where(pos, tm, t0)
        v = a * (1.0 - w) + b * w
        for q in range(8):
            s8_v[pl.ds(q * ROWSTRIDE + PAD + k0 + q - 7, L)] = v
        return carry

    lax.fori_loop(0, NCHUNK, chunk, 0)

    r0 = c * (NR // 2)

    def fire(j, carry):
        r = r0 + j
        a = r // 8
        q = r % 8
        pltpu.async_copy(
            s8_v.at[pl.ds(q * ROWSTRIDE + PAD + 120 - 8 * a, W)],
            s128_hbm.at[pl.ds((head * NR + r) * W, W)],
            sem,
        )
        return carry

    lax.fori_loop(0, NR // 2, fire, 0)

    def drain(j, carry):
        r = r0 + j
        a = r // 8
        q = r % 8
        pltpu.make_async_copy(
            s8_v.at[pl.ds(q * ROWSTRIDE + PAD + 120 - 8 * a, W)],
            s128_hbm.at[pl.ds((head * NR + r) * W, W)],
            sem,
        ).wait()
        return carry

    lax.fori_loop(0, NR // 2, drain, 0)


_sc_interp = functools.partial(
    pl.kernel,
    out_type=jax.ShapeDtypeStruct((H * NR * W,), jnp.float32),
    mesh=plsc.VectorSubcoreMesh(core_axis_name="c", subcore_axis_name="s"),
    scratch_types=[
        pltpu.VMEM((KTOT + 2 * PAD,), jnp.float32),
        pltpu.VMEM((8 * ROWSTRIDE,), jnp.float32),
        pltpu.VMEM((L,), jnp.float32),
        pltpu.SemaphoreType.DMA,
    ],
)(_sc_body)


def _tc_body(s_ref, out_ref, sem):
    h = pl.program_id(0)
    copies = []
    for g in range(T // NR):
        cp = pltpu.make_async_copy(
            s_ref.at[0, :, pl.ds(1920 - NR * g, T)],
            out_ref.at[0, h, pl.ds(NR * g, NR), :],
            sem,
        )
        cp.start()
        copies.append(cp)
    for cp in copies:
        cp.wait()


def _tc_expand(s128):
    return pl.pallas_call(
        _tc_body,
        grid=(H,),
        in_specs=[pl.BlockSpec((1, NR, W), lambda h: (h, 0, 0))],
        out_specs=pl.BlockSpec(memory_space=pltpu.MemorySpace.HBM),
        out_shape=jax.ShapeDtypeStruct((1, H, T, T), jnp.float32),
        scratch_shapes=[pltpu.SemaphoreType.DMA],
    )(s128)


def kernel(relative_position_bias_table, learnable_offset):
    tbl = relative_position_bias_table
    trp = jnp.asarray(
        jnp.concatenate([tbl, tbl[:1]], axis=0).T, jnp.float32)
    off16 = jnp.broadcast_to(learnable_offset.astype(jnp.float32), (L,))
    s128 = _sc_interp(trp.reshape(-1), off16)
    return _tc_expand(s128.reshape(H, NR, W))


# speedup vs baseline: 3.4006x; 1.0748x over previous
import functools

import jax
import jax.numpy as jnp
from jax import lax
from jax.experimental import pallas as pl
from jax.experimental.pallas import tpu as pltpu
from jax.experimental.pallas import tpu_sc as plsc

T = 2048
H = 16
L = 16
PAD = 16
KTOT = 2 * T
NCHUNK = KTOT // L
ROWSTRIDE = KTOT + PAD
W = 3968
NR = 128
BR = 256
NB = T // BR


def _sc_body(tr_hbm, off_hbm, s128_hbm, tr_v, s8_v, off_v, sem):
    c = lax.axis_index("c")
    s = lax.axis_index("s")
    head = s

    pltpu.sync_copy(tr_hbm.at[pl.ds(head * KTOT, KTOT)],
                    tr_v.at[pl.ds(PAD, KTOT)])
    pltpu.sync_copy(off_hbm, off_v)
    tr_v[pl.ds(0, L)] = lax.rev(tr_v[pl.ds(PAD, L)], (0,))

    x = off_v[...]
    e = jnp.exp(x + x)
    bo = (1.0 - 2.0 / (e + 1.0)) * 0.5
    pos = bo >= 0.0

    def chunk(i, carry):
        k0 = i * L
        kk = lax.iota(jnp.int32, L) + k0
        d = 4094.0 - kk.astype(jnp.float32)
        adj = jnp.clip(d + bo, 0.0, 4094.0)
        fl = adj.astype(jnp.int32).astype(jnp.float32)
        w = adj - fl
        t0 = lax.rev(tr_v[pl.ds(PAD + 4079 - k0, L)], (0,))
        tl = lax.rev(tr_v[pl.ds(PAD + 4078 - k0, L)], (0,))
        tm = lax.rev(tr_v[pl.ds(PAD + 4080 - k0, L)], (0,))
        a = jnp.where(pos, t0, tl)
        b = jnp.where(pos, tm, t0)
        v = a * (1.0 - w) + b * w
        for q in range(8):
            s8_v[pl.ds(q * ROWSTRIDE + PAD + k0 + q - 7, L)] = v
        return carry

    lax.fori_loop(0, NCHUNK, chunk, 0)

    r0 = c * (NR // 2)

    def fire(j, carry):
        r = r0 + j
        a = r // 8
        q = r % 8
        pltpu.async_copy(
            s8_v.at[pl.ds(q * ROWSTRIDE + PAD + 120 - 8 * a, W)],
            s128_hbm.at[pl.ds((head * NR + r) * W, W)],
            sem,
        )
        return carry

    lax.fori_loop(0, NR // 2, fire, 0)

    def drain(j, carry):
        r = r0 + j
        a = r // 8
        q = r % 8
        pltpu.make_async_copy(
            s8_v.at[pl.ds(q * ROWSTRIDE + PAD + 120 - 8 * a, W)],
            s128_hbm.at[pl.ds((head * NR + r) * W, W)],
            sem,
        ).wait()
        return carry

    lax.fori_loop(0, NR // 2, drain, 0)


_sc_interp = functools.partial(
    pl.kernel,
    out_type=jax.ShapeDtypeStruct((H * NR * W,), jnp.float32),
    mesh=plsc.VectorSubcoreMesh(core_axis_name="c", subcore_axis_name="s"),
    scratch_types=[
        pltpu.VMEM((KTOT + 2 * PAD,), jnp.float32),
        pltpu.VMEM((8 * ROWSTRIDE,), jnp.float32),
        pltpu.VMEM((L,), jnp.float32),
        pltpu.SemaphoreType.DMA,
    ],
)(_sc_body)


HB = 4


def _tc_body(s_ref, out_ref, sem):
    b = pl.program_id(0)
    copies = []
    for hl in range(HB):
        for g in range(T // NR):
            cp = pltpu.make_async_copy(
                s_ref.at[hl, :, pl.ds(1920 - NR * g, T)],
                out_ref.at[0, HB * b + hl, pl.ds(NR * g, NR), :],
                sem,
            )
            cp.start()
            copies.append(cp)
    for cp in copies:
        cp.wait()


def _tc_expand(s128):
    return pl.pallas_call(
        _tc_body,
        grid=(H // HB,),
        in_specs=[pl.BlockSpec((HB, NR, W), lambda b: (b, 0, 0))],
        out_specs=pl.BlockSpec(memory_space=pltpu.MemorySpace.HBM),
        out_shape=jax.ShapeDtypeStruct((1, H, T, T), jnp.float32),
        scratch_shapes=[pltpu.SemaphoreType.DMA],
    )(s128)


def kernel(relative_position_bias_table, learnable_offset):
    tbl = relative_position_bias_table
    trp = jnp.asarray(
        jnp.concatenate([tbl, tbl[:1]], axis=0).T, jnp.float32)
    off16 = jnp.broadcast_to(learnable_offset.astype(jnp.float32), (L,))
    s128 = _sc_interp(trp.reshape(-1), off16)
    return _tc_expand(s128.reshape(H, NR, W))


# speedup vs baseline: 3.4210x; 1.0060x over previous
import functools

import jax
import jax.numpy as jnp
from jax import lax
from jax.experimental import pallas as pl
from jax.experimental.pallas import tpu as pltpu
from jax.experimental.pallas import tpu_sc as plsc

T = 2048
H = 16
L = 16
PAD = 16
KTOT = 2 * T
NCHUNK = KTOT // L
ROWSTRIDE = KTOT + PAD
W = 3968
NR = 128
BR = 256
NB = T // BR


def _sc_body(tr_hbm, off_hbm, s128_hbm, tr_v, s8_v, off_v, sem):
    c = lax.axis_index("c")
    s = lax.axis_index("s")
    head = s

    pltpu.sync_copy(tr_hbm.at[pl.ds(head * KTOT, KTOT)],
                    tr_v.at[pl.ds(PAD, KTOT)])
    pltpu.sync_copy(off_hbm, off_v)
    tr_v[pl.ds(0, L)] = lax.rev(tr_v[pl.ds(PAD, L)], (0,))

    x = off_v[...]
    e = jnp.exp(x + x)
    bo = (1.0 - 2.0 / (e + 1.0)) * 0.5
    pos = bo >= 0.0

    def chunk(i, carry):
        k0 = i * L
        kk = lax.iota(jnp.int32, L) + k0
        d = 4094.0 - kk.astype(jnp.float32)
        adj = jnp.clip(d + bo, 0.0, 4094.0)
        fl = adj.astype(jnp.int32).astype(jnp.float32)
        w = adj - fl
        t0 = lax.rev(tr_v[pl.ds(PAD + 4079 - k0, L)], (0,))
        tl = lax.rev(tr_v[pl.ds(PAD + 4078 - k0, L)], (0,))
        tm = lax.rev(tr_v[pl.ds(PAD + 4080 - k0, L)], (0,))
        a = jnp.where(pos, t0, tl)
        b = jnp.where(pos, tm, t0)
        v = a * (1.0 - w) + b * w
        for q in range(8):
            s8_v[pl.ds(q * ROWSTRIDE + PAD + k0 + q - 7, L)] = v
        return carry

    lax.fori_loop(0, NCHUNK, chunk, 0)

    r0 = c * (NR // 2)

    def fire(j, carry):
        r = r0 + j
        a = r // 8
        q = r % 8
        pltpu.async_copy(
            s8_v.at[pl.ds(q * ROWSTRIDE + PAD + 120 - 8 * a, W)],
            s128_hbm.at[pl.ds((head * NR + r) * W, W)],
            sem,
        )
        return carry

    lax.fori_loop(0, NR // 2, fire, 0)

    def drain(j, carry):
        r = r0 + j
        a = r // 8
        q = r % 8
        pltpu.make_async_copy(
            s8_v.at[pl.ds(q * ROWSTRIDE + PAD + 120 - 8 * a, W)],
            s128_hbm.at[pl.ds((head * NR + r) * W, W)],
            sem,
        ).wait()
        return carry

    lax.fori_loop(0, NR // 2, drain, 0)


_sc_interp = functools.partial(
    pl.kernel,
    out_type=jax.ShapeDtypeStruct((H * NR * W,), jnp.float32),
    mesh=plsc.VectorSubcoreMesh(core_axis_name="c", subcore_axis_name="s"),
    scratch_types=[
        pltpu.VMEM((KTOT + 2 * PAD,), jnp.float32),
        pltpu.VMEM((8 * ROWSTRIDE,), jnp.float32),
        pltpu.VMEM((L,), jnp.float32),
        pltpu.SemaphoreType.DMA,
    ],
)(_sc_body)


HB = 8


def _tc_body(s_ref, out_ref, sem):
    b = pl.program_id(0)
    copies = []
    for hl in range(HB):
        for g in range(T // NR):
            cp = pltpu.make_async_copy(
                s_ref.at[hl, :, pl.ds(1920 - NR * g, T)],
                out_ref.at[0, HB * b + hl, pl.ds(NR * g, NR), :],
                sem,
            )
            cp.start()
            copies.append(cp)
    for cp in copies:
        cp.wait()


def _tc_expand(s128):
    return pl.pallas_call(
        _tc_body,
        grid=(H // HB,),
        in_specs=[pl.BlockSpec((HB, NR, W), lambda b: (b, 0, 0))],
        out_specs=pl.BlockSpec(memory_space=pltpu.MemorySpace.HBM),
        out_shape=jax.ShapeDtypeStruct((1, H, T, T), jnp.float32),
        scratch_shapes=[pltpu.SemaphoreType.DMA],
    )(s128)


def kernel(relative_position_bias_table, learnable_offset):
    tbl = relative_position_bias_table
    trp = jnp.asarray(
        jnp.concatenate([tbl, tbl[:1]], axis=0).T, jnp.float32)
    off16 = jnp.broadcast_to(learnable_offset.astype(jnp.float32), (L,))
    s128 = _sc_interp(trp.reshape(-1), off16)
    return _tc_expand(s128.reshape(H, NR, W))
